# Initial kernel scaffold; baseline (speedup 1.0000x reference)
import jax
import jax.numpy as jnp
from jax.experimental import pallas as pl


def kernel(x, edge_index_p, edge_index_s, edge_index_v, W_p, b_p, W_s, b_s, W_v, b_v, Wp1, bp1, Wp2, bp2, Wc1, bc1, Wc2, bc2):
    raise NotImplementedError("write your pallas kernel here")


# speedup vs baseline: 54.3002x; 54.3002x over previous
import functools

import jax
import jax.numpy as jnp
from jax import lax
from jax.experimental import pallas as pl
from jax.experimental.pallas import tpu as pltpu
from jax.experimental.pallas import tpu_sc as plsc

N = 100000
NPAD = 102400
E = 3200000
EPAD = 3211264
ROWS = EPAD // 128
RPW = ROWS // 32
RBLK = 8
NBLKS = RPW // RBLK
STRIPE = NPAD // 16
BLK = 2048
F32 = jnp.float32
I32 = jnp.int32

_MESH = plsc.VectorSubcoreMesh(core_axis_name="c", subcore_axis_name="s")
_SC_PARAMS = pltpu.CompilerParams(use_tc_tiling_on_sc=False)


def _leaky(v):
    return jnp.where(v >= 0, v, v * 0.1)


@functools.partial(
    pl.kernel,
    out_type=jax.ShapeDtypeStruct((6 * NPAD,), F32),
    mesh=_MESH,
    compiler_params=_SC_PARAMS,
    scratch_types=[
        pltpu.VMEM((RBLK, 128), I32),
        pltpu.VMEM((128,), F32),
        pltpu.VMEM((STRIPE,), F32),
        pltpu.VMEM_SHARED((NPAD,), F32),
        pltpu.VMEM_SHARED((NPAD,), F32),
        pltpu.VMEM_SHARED((NPAD,), F32),
    ],
)
def _degree_sc(dst_p, dst_s, dst_v, zeros1, out, idx_v, ones_v, buf_v,
               deg_p, deg_s, deg_v):
    c = lax.axis_index("c")
    s = lax.axis_index("s")
    wid = s * 2 + c
    base_row = wid * RPW

    for k in range(8):
        ones_v[pl.ds(k * 16, 16)] = jnp.ones((16,), F32)
    pltpu.sync_copy(zeros1, buf_v)
    stripe = pl.ds(s * STRIPE, STRIPE)
    for deg in (deg_p, deg_s, deg_v):
        pltpu.sync_copy(buf_v, deg.at[stripe])
    plsc.subcore_barrier()

    for dst, deg in ((dst_p, deg_p), (dst_s, deg_s), (dst_v, deg_v)):
        def body(g, _):
            pltpu.sync_copy(dst.at[pl.ds(base_row + g * RBLK, RBLK)], idx_v)
            for j in range(RBLK):
                pltpu.sync_copy(ones_v, deg.at[idx_v.at[j]], add=True)
            return 0
        lax.fori_loop(0, NBLKS, body, 0)

    plsc.subcore_barrier()
    for t, deg in enumerate((deg_p, deg_s, deg_v)):
        pltpu.sync_copy(deg.at[stripe], buf_v)
        off = (c * 3 + t) * NPAD + s * STRIPE
        pltpu.sync_copy(buf_v, out.at[pl.ds(off, STRIPE)])


@functools.partial(
    pl.kernel,
    out_type=jax.ShapeDtypeStruct((6 * NPAD, 8), F32),
    mesh=_MESH,
    compiler_params=_SC_PARAMS,
    scratch_types=[
        pltpu.VMEM((RBLK, 128), I32),
        pltpu.VMEM((RBLK, 128), I32),
        pltpu.VMEM((128, 8), F32),
        pltpu.VMEM((STRIPE, 8), F32),
        pltpu.VMEM_SHARED((NPAD, 8), F32),
    ],
)
def _message_sc(y_p, y_s, y_v, src_p, dst_p, src_s, dst_s, src_v, dst_v,
                zeros8, out, sidx, didx, rows_v, buf_v, acc):
    c = lax.axis_index("c")
    s = lax.axis_index("s")
    wid = s * 2 + c
    base_row = wid * RPW
    stripe = pl.ds(s * STRIPE, STRIPE)

    for t, (y, src, dst) in enumerate(((y_p, src_p, dst_p),
                                       (y_s, src_s, dst_s),
                                       (y_v, src_v, dst_v))):
        pltpu.sync_copy(zeros8, buf_v)
        pltpu.sync_copy(buf_v, acc.at[stripe])
        plsc.subcore_barrier()

        def body(g, _):
            rbase = pl.ds(base_row + g * RBLK, RBLK)
            pltpu.sync_copy(src.at[rbase], sidx)
            pltpu.sync_copy(dst.at[rbase], didx)
            for j in range(RBLK):
                pltpu.sync_copy(y.at[sidx.at[j]], rows_v)
                pltpu.sync_copy(rows_v, acc.at[didx.at[j]], add=True)
            return 0
        lax.fori_loop(0, NBLKS, body, 0)

        plsc.subcore_barrier()
        pltpu.sync_copy(acc.at[stripe], buf_v)
        off = (c * 3 + t) * NPAD + s * STRIPE
        pltpu.sync_copy(buf_v, out.at[pl.ds(off, STRIPE)])


def _build_y_body(degp_ref, x_ref, w_ref, y_ref, dinv_ref, xw_ref):
    xw = jnp.dot(x_ref[...], w_ref[...], preferred_element_type=F32)
    deg = degp_ref[0] + degp_ref[1] + 1.0
    dinv = lax.rsqrt(deg)
    dinv_ref[...] = dinv
    xw_ref[...] = xw
    y_ref[...] = jnp.stack(
        [dinv[t][:, None] * xw[:, 8 * t:8 * t + 8] for t in range(3)], axis=0)


def _build_y(degp, xpad, w3):
    grid = (NPAD // BLK,)
    return pl.pallas_call(
        _build_y_body,
        grid=grid,
        in_specs=[
            pl.BlockSpec((2, 3, BLK), lambda i: (0, 0, i)),
            pl.BlockSpec((BLK, 32), lambda i: (i, 0)),
            pl.BlockSpec((32, 24), lambda i: (0, 0)),
        ],
        out_specs=[
            pl.BlockSpec((3, BLK, 8), lambda i: (0, i, 0)),
            pl.BlockSpec((3, BLK), lambda i: (0, i)),
            pl.BlockSpec((BLK, 24), lambda i: (i, 0)),
        ],
        out_shape=[
            jax.ShapeDtypeStruct((3, NPAD, 8), F32),
            jax.ShapeDtypeStruct((3, NPAD), F32),
            jax.ShapeDtypeStruct((NPAD, 24), F32),
        ],
    )(degp, xpad, w3)


def _final_body(accp_ref, dinv_ref, xw_ref, b3_ref, wp1_ref, bp1_ref,
                wp2_ref, bp2_ref, wc1_ref, bc1_ref, wc2_ref, bc2_ref,
                out_ref):
    acc = accp_ref[0] + accp_ref[1]
    dinv = dinv_ref[...]
    xw = xw_ref[...]
    hs = []
    for t in range(3):
        g = (dinv[t][:, None] * acc[t]
             + (dinv[t] * dinv[t])[:, None] * xw[:, 8 * t:8 * t + 8]
             + b3_ref[t][None, :])
        hs.append(g)
    h = _leaky(jnp.concatenate(hs, axis=1))
    h1 = _leaky(jnp.dot(h, wp1_ref[...], preferred_element_type=F32)
                + bp1_ref[...])
    h2 = jnp.dot(h1, wp2_ref[...], preferred_element_type=F32) + bp2_ref[...]
    h3 = _leaky(jnp.dot(h2, wc1_ref[...], preferred_element_type=F32)
                + bc1_ref[...])
    out_ref[...] = (jnp.dot(h3, wc2_ref[...], preferred_element_type=F32)
                    + bc2_ref[...])


def _final(accp, dinv, xw, b3, wp1, bp1, wp2, bp2, wc1, bc1, wc2, bc2):
    grid = (NPAD // BLK,)
    full = lambda shape: pl.BlockSpec(shape, lambda i: tuple(0 for _ in shape))
    return pl.pallas_call(
        _final_body,
        grid=grid,
        in_specs=[
            pl.BlockSpec((2, 3, BLK, 8), lambda i: (0, 0, i, 0)),
            pl.BlockSpec((3, BLK), lambda i: (0, i)),
            pl.BlockSpec((BLK, 24), lambda i: (i, 0)),
            full((3, 8)),
            full((24, 16)), full((1, 16)),
            full((16, 8)), full((1, 8)),
            full((8, 8)), full((1, 8)),
            full((8, 8)), full((1, 8)),
        ],
        out_specs=pl.BlockSpec((BLK, 8), lambda i: (i, 0)),
        out_shape=jax.ShapeDtypeStruct((NPAD, 8), F32),
    )(accp, dinv, xw, b3, wp1, bp1, wp2, bp2, wc1, bc1, wc2, bc2)


def kernel(x, edge_index_p, edge_index_s, edge_index_v,
           W_p, b_p, W_s, b_s, W_v, b_v,
           Wp1, bp1, Wp2, bp2, Wc1, bc1, Wc2, bc2):
    pad = (N + (jnp.arange(EPAD - E, dtype=I32) % 2048)).astype(I32)

    def prep(ei):
        s = jnp.concatenate([ei[0].astype(I32), pad]).reshape(ROWS, 128)
        d = jnp.concatenate([ei[1].astype(I32), pad]).reshape(ROWS, 128)
        return s, d

    sp, dp = prep(edge_index_p)
    ss, ds_ = prep(edge_index_s)
    sv, dv = prep(edge_index_v)

    xpad = jnp.zeros((NPAD, 32), F32).at[:N, :25].set(x.astype(F32))
    w3 = jnp.zeros((32, 24), F32)
    for t, W in enumerate((W_p, W_s, W_v)):
        w3 = w3.at[:25, 8 * t:8 * t + 5].set(W.T)
    b3 = jnp.zeros((3, 8), F32).at[:, :5].set(jnp.stack((b_p, b_s, b_v)))

    wp1 = jnp.zeros((24, 16), F32)
    for t in range(3):
        wp1 = wp1.at[8 * t:8 * t + 5, :10].set(Wp1[:, 5 * t:5 * t + 5].T)
    bp1p = jnp.zeros((1, 16), F32).at[0, :10].set(bp1)
    wp2 = jnp.zeros((16, 8), F32).at[:10, :5].set(Wp2.T)
    bp2p = jnp.zeros((1, 8), F32).at[0, :5].set(bp2)
    wc1 = jnp.zeros((8, 8), F32).at[:5, :5].set(Wc1.T)
    bc1p = jnp.zeros((1, 8), F32).at[0, :5].set(bc1)
    wc2 = jnp.zeros((8, 8), F32).at[:5, :2].set(Wc2.T)
    bc2p = jnp.zeros((1, 8), F32).at[0, :2].set(bc2)

    zeros1 = jnp.zeros((STRIPE,), F32)
    zeros8 = jnp.zeros((STRIPE, 8), F32)

    degp = _degree_sc(dp, ds_, dv, zeros1).reshape(2, 3, NPAD)
    y, dinv, xw = _build_y(degp, xpad, w3)
    accp = _message_sc(y[0], y[1], y[2], sp, dp, ss, ds_, sv, dv,
                       zeros8).reshape(2, 3, NPAD, 8)
    out = _final(accp, dinv, xw, b3, wp1, bp1p, wp2, bp2p,
                 wc1, bc1p, wc2, bc2p)
    return out[:N, :2]


# speedup vs baseline: 116.9303x; 2.1534x over previous
import functools

import jax
import jax.numpy as jnp
from jax import lax
from jax.experimental import pallas as pl
from jax.experimental.pallas import tpu as pltpu
from jax.experimental.pallas import tpu_sc as plsc

N = 100000
NPAD = 102400
E = 3200000
EPAD = 3211264
ROWS = EPAD // 128
RPW = ROWS // 32
RBLK = 8
NBLKS = RPW // RBLK
STRIPE = NPAD // 16
BLK = 2048
F32 = jnp.float32
I32 = jnp.int32

_MESH = plsc.VectorSubcoreMesh(core_axis_name="c", subcore_axis_name="s")
_SC_PARAMS = pltpu.CompilerParams(use_tc_tiling_on_sc=False)


def _leaky(v):
    return jnp.where(v >= 0, v, v * 0.1)


@functools.partial(
    pl.kernel,
    out_type=jax.ShapeDtypeStruct((6 * NPAD,), F32),
    mesh=_MESH,
    compiler_params=_SC_PARAMS,
    scratch_types=[
        pltpu.VMEM((2, RBLK, 128), I32),
        pltpu.VMEM((128,), F32),
        pltpu.VMEM((STRIPE,), F32),
        pltpu.VMEM_SHARED((NPAD,), F32),
        pltpu.VMEM_SHARED((NPAD,), F32),
        pltpu.VMEM_SHARED((NPAD,), F32),
        pltpu.SemaphoreType.DMA,
        pltpu.SemaphoreType.DMA,
    ],
)
def _degree_sc(dst_p, dst_s, dst_v, zeros1, out, idx_v, ones_v, buf_v,
               deg_p, deg_s, deg_v, semi, sems):
    c = lax.axis_index("c")
    s = lax.axis_index("s")
    wid = s * 2 + c
    base_row = wid * RPW

    for k in range(8):
        ones_v[pl.ds(k * 16, 16)] = jnp.ones((16,), F32)
    pltpu.sync_copy(zeros1, buf_v)
    stripe = pl.ds(s * STRIPE, STRIPE)
    for deg in (deg_p, deg_s, deg_v):
        pltpu.sync_copy(buf_v, deg.at[stripe])
    plsc.subcore_barrier()

    for dst, deg in ((dst_p, deg_p), (dst_s, deg_s), (dst_v, deg_v)):
        pltpu.async_copy(dst.at[pl.ds(base_row, RBLK)], idx_v.at[0], semi)

        def body(g, _):
            cur = lax.rem(g, 2)
            nxt = lax.rem(g + 1, 2)
            pltpu.make_async_copy(
                dst.at[pl.ds(base_row, RBLK)], idx_v.at[cur], semi).wait()
            gn = jnp.minimum(g + 1, NBLKS - 1)
            pltpu.async_copy(
                dst.at[pl.ds(base_row + gn * RBLK, RBLK)], idx_v.at[nxt],
                semi)
            @pl.when(g > 0)
            def _():
                for j in range(RBLK):
                    pltpu.make_async_copy(
                        ones_v, deg.at[idx_v.at[cur, j]], sems).wait()
            for j in range(RBLK):
                pltpu.async_copy(ones_v, deg.at[idx_v.at[cur, j]], sems,
                                 add=True)
            return 0
        lax.fori_loop(0, NBLKS, body, 0)
        last = lax.rem(NBLKS - 1, 2)
        for j in range(RBLK):
            pltpu.make_async_copy(
                ones_v, deg.at[idx_v.at[last, j]], sems).wait()
        pltpu.make_async_copy(
            dst.at[pl.ds(base_row, RBLK)], idx_v.at[0], semi).wait()

    plsc.subcore_barrier()
    for t, deg in enumerate((deg_p, deg_s, deg_v)):
        pltpu.sync_copy(deg.at[stripe], buf_v)
        off = (c * 3 + t) * NPAD + s * STRIPE
        pltpu.sync_copy(buf_v, out.at[pl.ds(off, STRIPE)])


@functools.partial(
    pl.kernel,
    out_type=jax.ShapeDtypeStruct((6 * NPAD, 8), F32),
    mesh=_MESH,
    compiler_params=_SC_PARAMS,
    scratch_types=[
        pltpu.VMEM((3, RBLK, 128), I32),
        pltpu.VMEM((3, RBLK, 128), I32),
        pltpu.VMEM((2, RBLK, 128, 8), F32),
        pltpu.VMEM((STRIPE, 8), F32),
        pltpu.VMEM_SHARED((NPAD, 8), F32),
        pltpu.SemaphoreType.DMA,
        pltpu.SemaphoreType.DMA,
        pltpu.SemaphoreType.DMA,
    ],
)
def _message_sc(y_p, y_s, y_v, src_p, dst_p, src_s, dst_s, src_v, dst_v,
                zeros8, out, sidx, didx, rows_v, buf_v, acc,
                semi, semg, sems):
    c = lax.axis_index("c")
    s = lax.axis_index("s")
    wid = s * 2 + c
    base_row = wid * RPW
    stripe = pl.ds(s * STRIPE, STRIPE)

    for t, (y, src, dst) in enumerate(((y_p, src_p, dst_p),
                                       (y_s, src_s, dst_s),
                                       (y_v, src_v, dst_v))):
        pltpu.sync_copy(zeros8, buf_v)
        pltpu.sync_copy(buf_v, acc.at[stripe])
        plsc.subcore_barrier()

        rb0 = pl.ds(base_row, RBLK)
        pltpu.async_copy(src.at[rb0], sidx.at[0], semi)
        pltpu.async_copy(dst.at[rb0], didx.at[0], semi)

        def body(g, _):
            cur3 = lax.rem(g, 3)
            nxt3 = lax.rem(g + 1, 3)
            curr = lax.rem(g, 2)

            @pl.when(g > 1)
            def _():
                for j in range(RBLK):
                    pltpu.make_async_copy(
                        rows_v.at[curr, j], acc.at[didx.at[cur3, j]],
                        sems).wait()
            pltpu.make_async_copy(src.at[rb0], sidx.at[cur3], semi).wait()
            pltpu.make_async_copy(dst.at[rb0], didx.at[cur3], semi).wait()
            gn = jnp.minimum(g + 1, NBLKS - 1)
            rbn = pl.ds(base_row + gn * RBLK, RBLK)
            pltpu.async_copy(src.at[rbn], sidx.at[nxt3], semi)
            pltpu.async_copy(dst.at[rbn], didx.at[nxt3], semi)
            for j in range(RBLK):
                pltpu.async_copy(y.at[sidx.at[cur3, j]], rows_v.at[curr, j],
                                 semg)
            for j in range(RBLK):
                pltpu.make_async_copy(
                    y.at[sidx.at[cur3, j]], rows_v.at[curr, j], semg).wait()
            for j in range(RBLK):
                pltpu.async_copy(rows_v.at[curr, j],
                                 acc.at[didx.at[cur3, j]], sems, add=True)
            return 0
        lax.fori_loop(0, NBLKS, body, 0)

        for g in (NBLKS - 2, NBLKS - 1):
            for j in range(RBLK):
                pltpu.make_async_copy(
                    rows_v.at[g % 2, j], acc.at[didx.at[g % 3, j]],
                    sems).wait()
        pltpu.make_async_copy(src.at[rb0], sidx.at[0], semi).wait()
        pltpu.make_async_copy(dst.at[rb0], didx.at[0], semi).wait()

        plsc.subcore_barrier()
        pltpu.sync_copy(acc.at[stripe], buf_v)
        off = (c * 3 + t) * NPAD + s * STRIPE
        pltpu.sync_copy(buf_v, out.at[pl.ds(off, STRIPE)])


def _build_y_body(degp_ref, x_ref, w_ref, y_ref, dinv_ref, xw_ref):
    xw = jnp.dot(x_ref[...], w_ref[...], preferred_element_type=F32)
    deg = degp_ref[0] + degp_ref[1] + 1.0
    dinv = lax.rsqrt(deg)
    dinv_ref[...] = dinv
    xw_ref[...] = xw
    y_ref[...] = jnp.stack(
        [dinv[t][:, None] * xw[:, 8 * t:8 * t + 8] for t in range(3)], axis=0)


def _build_y(degp, xpad, w3):
    grid = (NPAD // BLK,)
    return pl.pallas_call(
        _build_y_body,
        grid=grid,
        in_specs=[
            pl.BlockSpec((2, 3, BLK), lambda i: (0, 0, i)),
            pl.BlockSpec((BLK, 32), lambda i: (i, 0)),
            pl.BlockSpec((32, 24), lambda i: (0, 0)),
        ],
        out_specs=[
            pl.BlockSpec((3, BLK, 8), lambda i: (0, i, 0)),
            pl.BlockSpec((3, BLK), lambda i: (0, i)),
            pl.BlockSpec((BLK, 24), lambda i: (i, 0)),
        ],
        out_shape=[
            jax.ShapeDtypeStruct((3, NPAD, 8), F32),
            jax.ShapeDtypeStruct((3, NPAD), F32),
            jax.ShapeDtypeStruct((NPAD, 24), F32),
        ],
    )(degp, xpad, w3)


def _final_body(accp_ref, dinv_ref, xw_ref, b3_ref, wp1_ref, bp1_ref,
                wp2_ref, bp2_ref, wc1_ref, bc1_ref, wc2_ref, bc2_ref,
                out_ref):
    acc = accp_ref[0] + accp_ref[1]
    dinv = dinv_ref[...]
    xw = xw_ref[...]
    hs = []
    for t in range(3):
        g = (dinv[t][:, None] * acc[t]
             + (dinv[t] * dinv[t])[:, None] * xw[:, 8 * t:8 * t + 8]
             + b3_ref[t][None, :])
        hs.append(g)
    h = _leaky(jnp.concatenate(hs, axis=1))
    h1 = _leaky(jnp.dot(h, wp1_ref[...], preferred_element_type=F32)
                + bp1_ref[...])
    h2 = jnp.dot(h1, wp2_ref[...], preferred_element_type=F32) + bp2_ref[...]
    h3 = _leaky(jnp.dot(h2, wc1_ref[...], preferred_element_type=F32)
                + bc1_ref[...])
    out_ref[...] = (jnp.dot(h3, wc2_ref[...], preferred_element_type=F32)
                    + bc2_ref[...])


def _final(accp, dinv, xw, b3, wp1, bp1, wp2, bp2, wc1, bc1, wc2, bc2):
    grid = (NPAD // BLK,)
    full = lambda shape: pl.BlockSpec(shape, lambda i: tuple(0 for _ in shape))
    return pl.pallas_call(
        _final_body,
        grid=grid,
        in_specs=[
            pl.BlockSpec((2, 3, BLK, 8), lambda i: (0, 0, i, 0)),
            pl.BlockSpec((3, BLK), lambda i: (0, i)),
            pl.BlockSpec((BLK, 24), lambda i: (i, 0)),
            full((3, 8)),
            full((24, 16)), full((1, 16)),
            full((16, 8)), full((1, 8)),
            full((8, 8)), full((1, 8)),
            full((8, 8)), full((1, 8)),
        ],
        out_specs=pl.BlockSpec((BLK, 8), lambda i: (i, 0)),
        out_shape=jax.ShapeDtypeStruct((NPAD, 8), F32),
    )(accp, dinv, xw, b3, wp1, bp1, wp2, bp2, wc1, bc1, wc2, bc2)


def kernel(x, edge_index_p, edge_index_s, edge_index_v,
           W_p, b_p, W_s, b_s, W_v, b_v,
           Wp1, bp1, Wp2, bp2, Wc1, bc1, Wc2, bc2):
    pad = (N + (jnp.arange(EPAD - E, dtype=I32) % 2048)).astype(I32)

    def prep(ei):
        s = jnp.concatenate([ei[0].astype(I32), pad]).reshape(ROWS, 128)
        d = jnp.concatenate([ei[1].astype(I32), pad]).reshape(ROWS, 128)
        return s, d

    sp, dp = prep(edge_index_p)
    ss, ds_ = prep(edge_index_s)
    sv, dv = prep(edge_index_v)

    xpad = jnp.zeros((NPAD, 32), F32).at[:N, :25].set(x.astype(F32))
    w3 = jnp.zeros((32, 24), F32)
    for t, W in enumerate((W_p, W_s, W_v)):
        w3 = w3.at[:25, 8 * t:8 * t + 5].set(W.T)
    b3 = jnp.zeros((3, 8), F32).at[:, :5].set(jnp.stack((b_p, b_s, b_v)))

    wp1 = jnp.zeros((24, 16), F32)
    for t in range(3):
        wp1 = wp1.at[8 * t:8 * t + 5, :10].set(Wp1[:, 5 * t:5 * t + 5].T)
    bp1p = jnp.zeros((1, 16), F32).at[0, :10].set(bp1)
    wp2 = jnp.zeros((16, 8), F32).at[:10, :5].set(Wp2.T)
    bp2p = jnp.zeros((1, 8), F32).at[0, :5].set(bp2)
    wc1 = jnp.zeros((8, 8), F32).at[:5, :5].set(Wc1.T)
    bc1p = jnp.zeros((1, 8), F32).at[0, :5].set(bc1)
    wc2 = jnp.zeros((8, 8), F32).at[:5, :2].set(Wc2.T)
    bc2p = jnp.zeros((1, 8), F32).at[0, :2].set(bc2)

    zeros1 = jnp.zeros((STRIPE,), F32)
    zeros8 = jnp.zeros((STRIPE, 8), F32)

    degp = _degree_sc(dp, ds_, dv, zeros1).reshape(2, 3, NPAD)
    y, dinv, xw = _build_y(degp, xpad, w3)
    accp = _message_sc(y[0], y[1], y[2], sp, dp, ss, ds_, sv, dv,
                       zeros8).reshape(2, 3, NPAD, 8)
    out = _final(accp, dinv, xw, b3, wp1, bp1p, wp2, bp2p,
                 wc1, bc1p, wc2, bc2p)
    return out[:N, :2]


# speedup vs baseline: 179.8242x; 1.5379x over previous
import functools

import jax
import jax.numpy as jnp
from jax import lax
from jax.experimental import pallas as pl
from jax.experimental.pallas import tpu as pltpu
from jax.experimental.pallas import tpu_sc as plsc

N = 100000
NPAD = 102400
E = 3200000
EROWS = E // 128
RBLK = 8
TBLKS = EROWS // RBLK
BASE_BLKS = TBLKS // 32
EXTRA = TBLKS - 32 * BASE_BLKS
STRIPE = NPAD // 16
BLK = 2048
PK = BLK // 16
F32 = jnp.float32
I32 = jnp.int32

_MESH = plsc.VectorSubcoreMesh(core_axis_name="c", subcore_axis_name="s")
_SC_PARAMS = pltpu.CompilerParams(use_tc_tiling_on_sc=False)


def _leaky(v):
    return jnp.where(v >= 0, v, v * 0.1)


def _worker_blocks(wid):
    base = wid * BASE_BLKS + jnp.minimum(wid, EXTRA)
    nblk = BASE_BLKS + jnp.where(wid < EXTRA, 1, 0)
    return base, nblk


@functools.partial(
    pl.kernel,
    out_type=jax.ShapeDtypeStruct((6 * NPAD,), F32),
    mesh=_MESH,
    compiler_params=_SC_PARAMS,
    scratch_types=[
        pltpu.VMEM((2, RBLK, 128), I32),
        pltpu.VMEM((128,), F32),
        pltpu.VMEM((STRIPE,), F32),
        pltpu.VMEM_SHARED((NPAD,), F32),
        pltpu.VMEM_SHARED((NPAD,), F32),
        pltpu.VMEM_SHARED((NPAD,), F32),
        pltpu.SemaphoreType.DMA,
        pltpu.SemaphoreType.DMA,
    ],
)
def _degree_sc(ei_p, ei_s, ei_v, zeros1, out, idx_v, ones_v, buf_v,
               deg_p, deg_s, deg_v, semi, sems):
    c = lax.axis_index("c")
    s = lax.axis_index("s")
    wid = s * 2 + c
    base_blk, nblk = _worker_blocks(wid)
    base_row = base_blk * RBLK

    for k in range(8):
        ones_v[pl.ds(k * 16, 16)] = jnp.ones((16,), F32)
    pltpu.sync_copy(zeros1, buf_v)
    stripe = pl.ds(s * STRIPE, STRIPE)
    for deg in (deg_p, deg_s, deg_v):
        pltpu.sync_copy(buf_v, deg.at[stripe])
    plsc.subcore_barrier()

    for ei, deg in ((ei_p, deg_p), (ei_s, deg_s), (ei_v, deg_v)):
        dst = ei.at[1]
        pltpu.async_copy(dst.at[pl.ds(base_row, RBLK)], idx_v.at[0], semi)

        def body(g, _):
            cur = lax.rem(g, 2)
            nxt = lax.rem(g + 1, 2)
            pltpu.make_async_copy(
                dst.at[pl.ds(base_row, RBLK)], idx_v.at[0], semi).wait()
            gn = jnp.minimum(g + 1, nblk - 1)
            pltpu.async_copy(
                dst.at[pl.ds(base_row + gn * RBLK, RBLK)], idx_v.at[nxt],
                semi)
            @pl.when(g > 0)
            def _():
                for j in range(RBLK):
                    pltpu.make_async_copy(
                        ones_v, deg.at[idx_v.at[0, j]], sems).wait()
            for j in range(RBLK):
                pltpu.async_copy(ones_v, deg.at[idx_v.at[cur, j]], sems,
                                 add=True)
            return 0
        lax.fori_loop(0, nblk, body, 0)
        for j in range(RBLK):
            pltpu.make_async_copy(
                ones_v, deg.at[idx_v.at[0, j]], sems).wait()
        pltpu.make_async_copy(
            dst.at[pl.ds(base_row, RBLK)], idx_v.at[0], semi).wait()

    plsc.subcore_barrier()
    for t, deg in enumerate((deg_p, deg_s, deg_v)):
        pltpu.sync_copy(deg.at[stripe], buf_v)
        off = (c * 3 + t) * NPAD + s * STRIPE
        pltpu.sync_copy(buf_v, out.at[pl.ds(off, STRIPE)])


@functools.partial(
    pl.kernel,
    out_type=jax.ShapeDtypeStruct((6 * NPAD, 8), F32),
    mesh=_MESH,
    compiler_params=_SC_PARAMS,
    scratch_types=[
        pltpu.VMEM((3, RBLK, 128), I32),
        pltpu.VMEM((3, RBLK, 128), I32),
        pltpu.VMEM((2, RBLK, 128, 8), F32),
        pltpu.VMEM((STRIPE, 8), F32),
        pltpu.VMEM_SHARED((NPAD, 8), F32),
        pltpu.SemaphoreType.DMA,
        pltpu.SemaphoreType.DMA,
        pltpu.SemaphoreType.DMA,
    ],
)
def _message_sc(y3, ei_p, ei_s, ei_v, zeros8, out, sidx, didx, rows_v,
                buf_v, acc, semi, semg, sems):
    c = lax.axis_index("c")
    s = lax.axis_index("s")
    wid = s * 2 + c
    base_blk, nblk = _worker_blocks(wid)
    base_row = base_blk * RBLK
    stripe = pl.ds(s * STRIPE, STRIPE)

    for t, ei in enumerate((ei_p, ei_s, ei_v)):
        y = y3.at[t]
        src = ei.at[0]
        dst = ei.at[1]
        pltpu.sync_copy(zeros8, buf_v)
        pltpu.sync_copy(buf_v, acc.at[stripe])
        plsc.subcore_barrier()

        rb0 = pl.ds(base_row, RBLK)
        pltpu.async_copy(src.at[rb0], sidx.at[0], semi)
        pltpu.async_copy(dst.at[rb0], didx.at[0], semi)

        def body(g, _):
            cur3 = lax.rem(g, 3)
            nxt3 = lax.rem(g + 1, 3)
            curr = lax.rem(g, 2)

            @pl.when(g > 1)
            def _():
                for j in range(RBLK):
                    pltpu.make_async_copy(
                        rows_v.at[0, j], acc.at[didx.at[0, j]], sems).wait()
            pltpu.make_async_copy(src.at[rb0], sidx.at[0], semi).wait()
            pltpu.make_async_copy(dst.at[rb0], didx.at[0], semi).wait()
            gn = jnp.minimum(g + 1, nblk - 1)
            rbn = pl.ds(base_row + gn * RBLK, RBLK)
            pltpu.async_copy(src.at[rbn], sidx.at[nxt3], semi)
            pltpu.async_copy(dst.at[rbn], didx.at[nxt3], semi)
            for j in range(RBLK):
                pltpu.async_copy(y.at[sidx.at[cur3, j]], rows_v.at[curr, j],
                                 semg)
            for j in range(RBLK):
                pltpu.make_async_copy(
                    y.at[sidx.at[cur3, j]], rows_v.at[curr, j], semg).wait()
            for j in range(RBLK):
                pltpu.async_copy(rows_v.at[curr, j],
                                 acc.at[didx.at[cur3, j]], sems, add=True)
            return 0
        lax.fori_loop(0, nblk, body, 0)

        for _ in range(2):
            for j in range(RBLK):
                pltpu.make_async_copy(
                    rows_v.at[0, j], acc.at[didx.at[0, j]], sems).wait()
        pltpu.make_async_copy(src.at[rb0], sidx.at[0], semi).wait()
        pltpu.make_async_copy(dst.at[rb0], didx.at[0], semi).wait()

        plsc.subcore_barrier()
        pltpu.sync_copy(acc.at[stripe], buf_v)
        off = (c * 3 + t) * NPAD + s * STRIPE
        pltpu.sync_copy(buf_v, out.at[pl.ds(off, STRIPE)])


def _build_y_body(degp_ref, xg_ref, wbig_ref, rep_ref, y_ref, dinvp_ref):
    xw = jnp.dot(xg_ref[...], wbig_ref[...], preferred_element_type=F32)
    deg = degp_ref[0] + degp_ref[1] + 1.0
    dinv = lax.rsqrt(deg)
    ys, ds = [], []
    for t in range(3):
        dp = jnp.dot(dinv[t], rep_ref[...], preferred_element_type=F32)
        ds.append(dp)
        ys.append(dp * xw[:, 128 * t:128 * (t + 1)])
    y_ref[...] = jnp.stack(ys, axis=0)
    dinvp_ref[...] = jnp.stack(ds, axis=0)


def _build_y(degp16, xg, wbig, rep):
    grid = (NPAD // BLK,)
    return pl.pallas_call(
        _build_y_body,
        grid=grid,
        in_specs=[
            pl.BlockSpec((2, 3, PK, 16), lambda i: (0, 0, i, 0)),
            pl.BlockSpec((PK, 512), lambda i: (i, 0)),
            pl.BlockSpec((512, 384), lambda i: (0, 0)),
            pl.BlockSpec((16, 128), lambda i: (0, 0)),
        ],
        out_specs=[
            pl.BlockSpec((3, PK, 128), lambda i: (0, i, 0)),
            pl.BlockSpec((3, PK, 128), lambda i: (0, i, 0)),
        ],
        out_shape=[
            jax.ShapeDtypeStruct((3, NPAD // 16, 128), F32),
            jax.ShapeDtypeStruct((3, NPAD // 16, 128), F32),
        ],
    )(degp16, xg, wbig, rep)


def _final_body(accp_ref, y_ref, dinvp_ref, b3_ref, m1_ref, bp1_ref,
                m2_ref, bp2_ref, m3_ref, bc1_ref, m4_ref, bc2_ref,
                out_ref):
    acc6 = accp_ref[...]
    yp = y_ref[...]
    dinvp = dinvp_ref[...]
    hs = [dinvp[t] * (acc6[t] + acc6[3 + t] + yp[t]) + b3_ref[t][None, :]
          for t in range(3)]
    h = _leaky(jnp.concatenate(hs, axis=1))
    h1 = _leaky(jnp.dot(h, m1_ref[...], preferred_element_type=F32)
                + bp1_ref[...])
    h2 = jnp.dot(h1, m2_ref[...], preferred_element_type=F32) + bp2_ref[...]
    h3 = _leaky(jnp.dot(h2, m3_ref[...], preferred_element_type=F32)
                + bc1_ref[...])
    out_ref[...] = (jnp.dot(h3, m4_ref[...], preferred_element_type=F32)
                    + bc2_ref[...])


def _final(accp, y, dinvp, b3t, m1, bp1t, m2, bp2t, m3, bc1t, m4, bc2t):
    grid = (NPAD // BLK,)
    full = lambda shape: pl.BlockSpec(shape, lambda i: tuple(0 for _ in shape))
    return pl.pallas_call(
        _final_body,
        grid=grid,
        in_specs=[
            pl.BlockSpec((6, PK, 128), lambda i: (0, i, 0)),
            pl.BlockSpec((3, PK, 128), lambda i: (0, i, 0)),
            pl.BlockSpec((3, PK, 128), lambda i: (0, i, 0)),
            full((3, 128)),
            full((384, 256)), full((1, 256)),
            full((256, 128)), full((1, 128)),
            full((128, 128)), full((1, 128)),
            full((128, 128)), full((1, 128)),
        ],
        out_specs=pl.BlockSpec((PK, 128), lambda i: (i, 0)),
        out_shape=jax.ShapeDtypeStruct((NPAD // 16, 128), F32),
    )(accp, y, dinvp, b3t, m1, bp1t, m2, bp2t, m3, bc1t, m4, bc2t)


def kernel(x, edge_index_p, edge_index_s, edge_index_v,
           W_p, b_p, W_s, b_s, W_v, b_v,
           Wp1, bp1, Wp2, bp2, Wc1, bc1, Wc2, bc2):
    ei_p = edge_index_p.astype(I32).reshape(2, EROWS, 128)
    ei_s = edge_index_s.astype(I32).reshape(2, EROWS, 128)
    ei_v = edge_index_v.astype(I32).reshape(2, EROWS, 128)

    xpad = jnp.zeros((NPAD, 32), F32).at[:N, :25].set(x.astype(F32))
    xg = xpad.reshape(NPAD // 16, 512)
    w3 = jnp.zeros((32, 24), F32)
    for t, W in enumerate((W_p, W_s, W_v)):
        w3 = w3.at[:25, 8 * t:8 * t + 5].set(W.T)
    eye = jnp.eye(16, dtype=F32)
    wbig = jnp.einsum("ic,ktj->iktcj", eye,
                      w3.reshape(32, 3, 8)).reshape(512, 384)
    rep = jnp.repeat(eye, 8, axis=1)
    b3 = jnp.zeros((3, 8), F32).at[:, :5].set(jnp.stack((b_p, b_s, b_v)))
    b3t = jnp.tile(b3, (1, 16))

    wp1 = jnp.zeros((24, 16), F32)
    for t in range(3):
        wp1 = wp1.at[8 * t:8 * t + 5, :10].set(Wp1[:, 5 * t:5 * t + 5].T)
    m1 = jnp.einsum("ik,tjo->tijko", eye,
                    wp1.reshape(3, 8, 16)).reshape(384, 256)
    bp1t = jnp.tile(jnp.zeros((1, 16), F32).at[0, :10].set(bp1), (1, 16))
    wp2 = jnp.zeros((16, 8), F32).at[:10, :5].set(Wp2.T)
    m2 = jnp.einsum("ij,ko->ikjo", eye, wp2).reshape(256, 128)
    bp2t = jnp.tile(jnp.zeros((1, 8), F32).at[0, :5].set(bp2), (1, 16))
    wc1 = jnp.zeros((8, 8), F32).at[:5, :5].set(Wc1.T)
    m3 = jnp.einsum("ij,ko->ikjo", eye, wc1).reshape(128, 128)
    bc1t = jnp.tile(jnp.zeros((1, 8), F32).at[0, :5].set(bc1), (1, 16))
    wc2 = jnp.zeros((8, 8), F32).at[:5, :2].set(Wc2.T)
    m4 = jnp.einsum("ij,ko->ikjo", eye, wc2).reshape(128, 128)
    bc2t = jnp.tile(jnp.zeros((1, 8), F32).at[0, :2].set(bc2), (1, 16))

    zeros1 = jnp.zeros((STRIPE,), F32)
    zeros8 = jnp.zeros((STRIPE, 8), F32)

    degp16 = _degree_sc(ei_p, ei_s, ei_v, zeros1).reshape(2, 3, NPAD // 16,
                                                          16)
    y, dinvp = _build_y(degp16, xg, wbig, rep)
    y3 = y.reshape(3, NPAD, 8)
    accp = _message_sc(y3, ei_p, ei_s, ei_v, zeros8)
    accp = accp.reshape(6, NPAD // 16, 128)
    out = _final(accp, y, dinvp, b3t, m1, bp1t, m2, bp2t, m3, bc1t, m4,
                 bc2t)
    return out.reshape(NPAD, 8)[:N, :2]
